# baseline (speedup 1.0000x reference)
import functools

import jax
import jax.numpy as jnp
from jax import lax
from jax.experimental import pallas as pl
from jax.experimental.pallas import tpu as pltpu
from jax.experimental.pallas import tpu_sc as plsc

_BATCH = 16384
_K = 64
_N = 100000
_W = 1536
_CH = 1024
_NRING = 16


def _nmf_body(u_hbm, i_hbm, pt_hbm, qt_hbm, rmp_hbm, rmq_hbm,
              win, win128, win32, ibuf, cbuf_id, cbuf_b, stage, fired_ref,
              sem_out, sem_in):
    cid = lax.axis_index("c")
    sid = lax.axis_index("s")
    wid = sid * 2 + cid
    lanes = lax.iota(jnp.int32, 16)

    def extract_lane(vec, m):
        return jnp.sum(jnp.where(lanes == m, vec, 0))

    def do_side(idx_hbm, rm_hbm, wref, wsize, start):
        fired_ref[0] = 0

        def chunk_loop(c, carry):
            pltpu.sync_copy(idx_hbm.at[pl.ds(c * _CH, _CH)], ibuf)

            def vec_loop(g, carry):
                v = ibuf[pl.ds(pl.multiple_of(g * 16, 16), 16)]
                m = (v >= start) & (v < start + wsize)
                n = jnp.sum(m.astype(jnp.int32))

                @pl.when(n > 0)
                def _():
                    bvec = c * _CH + g * 16 + lanes
                    plsc.store_compressed(cbuf_id.at[pl.ds(0, 16)], v, mask=m)
                    plsc.store_compressed(cbuf_b.at[pl.ds(0, 16)], bvec, mask=m)
                    cid_v = cbuf_id[pl.ds(0, 16)]
                    cb_v = cbuf_b[pl.ds(0, 16)]

                    def match_loop(mm, carry):
                        u = extract_lane(cid_v, mm)
                        b = extract_lane(cb_v, mm)
                        ul = u - start
                        f = fired_ref[0]
                        s = f % _NRING

                        @pl.when(f >= _NRING)
                        def _():
                            pltpu.make_async_copy(
                                stage.at[0], rm_hbm.at[0], sem_out).wait()

                        for cc in range(_K // 16):
                            colv = plsc.load_gather(
                                wref,
                                [lanes + 16 * cc,
                                 jnp.full((16,), 0, jnp.int32) + ul])
                            stage[s, pl.ds(16 * cc, 16)] = colv
                        pltpu.make_async_copy(
                            stage.at[s], rm_hbm.at[b], sem_out).start()
                        fired_ref[0] = f + 1
                        return carry

                    lax.fori_loop(0, n, match_loop, 0)

                return carry

            return lax.fori_loop(0, _CH // 16, vec_loop, carry)

        lax.fori_loop(0, _BATCH // _CH, chunk_loop, 0)
        nwait = jnp.minimum(fired_ref[0], _NRING)
        lax.fori_loop(
            0, nwait,
            lambda _, c: (pltpu.make_async_copy(
                stage.at[0], rm_hbm.at[0], sem_out).wait(), c)[1],
            0)

    def do_window(table_hbm, idx_hbm, rm_hbm, wref, wsize, start):
        pltpu.sync_copy(table_hbm.at[:, pl.ds(start, wsize)], wref)
        do_side(idx_hbm, rm_hbm, wref, wsize, start)

    for r in range(2):
        start = pl.multiple_of((wid + 32 * r) * _W, 128)
        do_window(pt_hbm, u_hbm, rmp_hbm, win, _W, start)
        do_window(qt_hbm, i_hbm, rmq_hbm, win, _W, start)

    @pl.when(wid == 0)
    def _():
        do_window(pt_hbm, u_hbm, rmp_hbm, win, _W, 64 * _W)
        do_window(qt_hbm, i_hbm, rmq_hbm, win, _W, 64 * _W)

    @pl.when(wid == 1)
    def _():
        do_window(pt_hbm, u_hbm, rmp_hbm, win128, 128, 99840)
        do_window(qt_hbm, i_hbm, rmq_hbm, win128, 128, 99840)

    @pl.when(wid == 2)
    def _():
        do_window(pt_hbm, u_hbm, rmp_hbm, win32, 32, 99968)
        do_window(qt_hbm, i_hbm, rmq_hbm, win32, 32, 99968)


_nmf_extract = functools.partial(
    pl.kernel,
    out_type=(jax.ShapeDtypeStruct((_BATCH, _K), jnp.float32),
              jax.ShapeDtypeStruct((_BATCH, _K), jnp.float32)),
    mesh=plsc.VectorSubcoreMesh(core_axis_name="c", subcore_axis_name="s"),
    compiler_params=pltpu.CompilerParams(needs_layout_passes=False),
    scratch_types=[
        pltpu.VMEM((_K, _W), jnp.float32),
        pltpu.VMEM((_K, 128), jnp.float32),
        pltpu.VMEM((_K, 32), jnp.float32),
        pltpu.VMEM((_CH,), jnp.int32),
        pltpu.VMEM((16,), jnp.int32),
        pltpu.VMEM((16,), jnp.int32),
        pltpu.VMEM((_NRING, _K), jnp.float32),
        pltpu.SMEM((1,), jnp.int32),
        pltpu.SemaphoreType.DMA,
        pltpu.SemaphoreType.DMA,
    ],
)(_nmf_body)


_DOTB = 1024


def _dot_body(p_ref, q_ref, o_ref):
    o_ref[...] = jnp.sum(p_ref[...] * q_ref[...], axis=1)


_dot_tc = pl.pallas_call(
    _dot_body,
    grid=(_BATCH // _DOTB,),
    in_specs=[
        pl.BlockSpec((_DOTB, _K), lambda i: (i, 0)),
        pl.BlockSpec((_DOTB, _K), lambda i: (i, 0)),
    ],
    out_specs=pl.BlockSpec((_DOTB,), lambda i: (i,)),
    out_shape=jax.ShapeDtypeStruct((_BATCH,), jnp.float32),
)


def kernel(train_x, P, Q):
    user_id = train_x[:, 0].astype(jnp.int32)
    item_id = train_x[:, 1].astype(jnp.int32)
    rmp, rmq = _nmf_extract(user_id, item_id, P.T, Q.T)
    return _dot_tc(rmp, rmq)


# speedup vs baseline: 2.0265x; 2.0265x over previous
import functools

import jax
import jax.numpy as jnp
from jax import lax
from jax.experimental import pallas as pl
from jax.experimental.pallas import tpu as pltpu
from jax.experimental.pallas import tpu_sc as plsc

_BATCH = 16384
_K = 64
_N = 100000
_RANGE = 3072
_W = 768
_CH = 1024
_NRING = 16


def _nmf_body(u_hbm, i_hbm, pt_hbm, qt_hbm, rmp_hbm, rmq_hbm,
              win, win128, win32, ibu, ibi, lu, li, cbuf, stage, fired_ref,
              sem_out, sem_in):
    cid = lax.axis_index("c")
    sid = lax.axis_index("s")
    wid = sid * 2 + cid
    lanes = lax.iota(jnp.int32, 16)

    lo = wid * _RANGE
    hi = lo + _RANGE
    lo2 = jnp.where(wid == 0, 98304,
                    jnp.where(wid == 1, 99072,
                              jnp.where(wid == 2, 99840, 0)))
    hi2 = jnp.where(wid == 0, 99072,
                    jnp.where(wid == 1, 99840,
                              jnp.where(wid == 2, _N, 0)))

    def scan_chunk(c, counts):
        pltpu.sync_copy(u_hbm.at[pl.ds(c * _CH, _CH)], ibu)
        pltpu.sync_copy(i_hbm.at[pl.ds(c * _CH, _CH)], ibi)

        def scan_vec(g, counts):
            cnt_u, cnt_i = counts
            bvec = c * _CH + g * 16 + lanes
            vu = ibu[pl.ds(pl.multiple_of(g * 16, 16), 16)]
            vi = ibi[pl.ds(pl.multiple_of(g * 16, 16), 16)]
            mu = ((vu >= lo) & (vu < hi)) | ((vu >= lo2) & (vu < hi2))
            mi = ((vi >= lo) & (vi < hi)) | ((vi >= lo2) & (vi < hi2))
            cu = jnp.cumsum(mu.astype(jnp.int32))
            ci = jnp.cumsum(mi.astype(jnp.int32))
            plsc.store_scatter(
                lu.at[pl.ds(0, _BATCH)], [cnt_u + cu - 1],
                (vu << 14) | bvec, mask=mu)
            plsc.store_scatter(
                li.at[pl.ds(0, _BATCH)], [cnt_i + ci - 1],
                (vi << 14) | bvec, mask=mi)
            return cnt_u + jnp.max(cu), cnt_i + jnp.max(ci)

        return lax.fori_loop(0, _CH // 16, scan_vec, counts)

    cnt_u, cnt_i = lax.fori_loop(0, _BATCH // _CH, scan_chunk, (0, 0))

    def do_side(lst, cnt, rm_hbm, wref, wsize, start):
        fired_ref[0] = 0

        def vec_loop(t, carry):
            pk = lst[pl.ds(pl.multiple_of(t * 16, 16), 16)]
            ids = pk >> 14
            m = (ids >= start) & (ids < start + wsize) \
                & (t * 16 + lanes < cnt)
            n = jnp.sum(m.astype(jnp.int32))

            @pl.when(n > 0)
            def _():
                plsc.store_compressed(cbuf.at[pl.ds(0, 16)], pk, mask=m)
                cb_v = cbuf[pl.ds(0, 16)]

                def match_loop(mm, carry):
                    e = jnp.sum(jnp.where(lanes == mm, cb_v, 0))
                    b = e & (_BATCH - 1)
                    ul = (e >> 14) - start
                    f = fired_ref[0]
                    s = f % _NRING

                    @pl.when(f >= _NRING)
                    def _():
                        pltpu.make_async_copy(
                            stage.at[0], rm_hbm.at[0], sem_out).wait()

                    for cc in range(_K // 16):
                        colv = plsc.load_gather(
                            wref,
                            [lanes + 16 * cc,
                             jnp.full((16,), 0, jnp.int32) + ul])
                        stage[s, pl.ds(16 * cc, 16)] = colv
                    pltpu.make_async_copy(
                        stage.at[s], rm_hbm.at[b], sem_out).start()
                    fired_ref[0] = f + 1
                    return carry

                lax.fori_loop(0, n, match_loop, 0)

            return carry

        lax.fori_loop(0, (cnt + 15) // 16, vec_loop, 0)
        nwait = jnp.minimum(fired_ref[0], _NRING)
        lax.fori_loop(
            0, nwait,
            lambda _, c: (pltpu.make_async_copy(
                stage.at[0], rm_hbm.at[0], sem_out).wait(), c)[1],
            0)

    def do_window(table_hbm, lst, cnt, rm_hbm, wref, wsize, start):
        pltpu.sync_copy(table_hbm.at[:, pl.ds(start, wsize)], wref)
        do_side(lst, cnt, rm_hbm, wref, wsize, start)

    for r in range(_RANGE // _W):
        start = pl.multiple_of(lo + r * _W, 128)
        do_window(pt_hbm, lu, cnt_u, rmp_hbm, win, _W, start)
        do_window(qt_hbm, li, cnt_i, rmq_hbm, win, _W, start)

    @pl.when(wid == 0)
    def _():
        do_window(pt_hbm, lu, cnt_u, rmp_hbm, win, _W, 98304)
        do_window(qt_hbm, li, cnt_i, rmq_hbm, win, _W, 98304)

    @pl.when(wid == 1)
    def _():
        do_window(pt_hbm, lu, cnt_u, rmp_hbm, win, _W, 99072)
        do_window(qt_hbm, li, cnt_i, rmq_hbm, win, _W, 99072)

    @pl.when(wid == 2)
    def _():
        do_window(pt_hbm, lu, cnt_u, rmp_hbm, win128, 128, 99840)
        do_window(qt_hbm, li, cnt_i, rmq_hbm, win128, 128, 99840)
        do_window(pt_hbm, lu, cnt_u, rmp_hbm, win32, 32, 99968)
        do_window(qt_hbm, li, cnt_i, rmq_hbm, win32, 32, 99968)


_nmf_extract = functools.partial(
    pl.kernel,
    out_type=(jax.ShapeDtypeStruct((_BATCH, _K), jnp.float32),
              jax.ShapeDtypeStruct((_BATCH, _K), jnp.float32)),
    mesh=plsc.VectorSubcoreMesh(core_axis_name="c", subcore_axis_name="s"),
    compiler_params=pltpu.CompilerParams(needs_layout_passes=False),
    scratch_types=[
        pltpu.VMEM((_K, _W), jnp.float32),
        pltpu.VMEM((_K, 128), jnp.float32),
        pltpu.VMEM((_K, 32), jnp.float32),
        pltpu.VMEM((_CH,), jnp.int32),
        pltpu.VMEM((_CH,), jnp.int32),
        pltpu.VMEM((_BATCH,), jnp.int32),
        pltpu.VMEM((_BATCH,), jnp.int32),
        pltpu.VMEM((16,), jnp.int32),
        pltpu.VMEM((_NRING, _K), jnp.float32),
        pltpu.SMEM((1,), jnp.int32),
        pltpu.SemaphoreType.DMA,
        pltpu.SemaphoreType.DMA,
    ],
)(_nmf_body)


_DOTB = 1024


def _dot_body(p_ref, q_ref, o_ref):
    o_ref[...] = jnp.sum(p_ref[...] * q_ref[...], axis=1)


_dot_tc = pl.pallas_call(
    _dot_body,
    grid=(_BATCH // _DOTB,),
    in_specs=[
        pl.BlockSpec((_DOTB, _K), lambda i: (i, 0)),
        pl.BlockSpec((_DOTB, _K), lambda i: (i, 0)),
    ],
    out_specs=pl.BlockSpec((_DOTB,), lambda i: (i,)),
    out_shape=jax.ShapeDtypeStruct((_BATCH,), jnp.float32),
)


def kernel(train_x, P, Q):
    user_id = train_x[:, 0].astype(jnp.int32)
    item_id = train_x[:, 1].astype(jnp.int32)
    rmp, rmq = _nmf_extract(user_id, item_id, P.T, Q.T)
    return _dot_tc(rmp, rmq)
